# Initial kernel scaffold; baseline (speedup 1.0000x reference)
import jax
import jax.numpy as jnp
from jax.experimental import pallas as pl


def kernel(edge_index, edge_weight, labels):
    raise NotImplementedError("write your pallas kernel here")


# speedup vs baseline: 99.0103x; 99.0103x over previous
import functools

import jax
import jax.numpy as jnp
from jax import lax
from jax.experimental import pallas as pl
from jax.experimental.pallas import tpu as pltpu
from jax.experimental.pallas import tpu_sc as plsc

K = 9
N_ROWS = 500000
N_NODES = 100000
L = 16
NC, NS = 2, 16
NW = NC * NS
ROWS_PER_BLOCK = L
EDGES_PER_BLOCK = ROWS_PER_BLOCK * K
BLOCKS_PER_CHUNK = 25
CHUNK_ROWS = BLOCKS_PER_CHUNK * ROWS_PER_BLOCK
CHUNK_EDGES = CHUNK_ROWS * K
N_CHUNKS = N_ROWS // CHUNK_ROWS
MAX_CHUNKS_PER_W = -(-N_CHUNKS // NW)

_mesh = plsc.VectorSubcoreMesh(core_axis_name="c", subcore_axis_name="s")


@functools.partial(
    pl.kernel,
    mesh=_mesh,
    compiler_params=pltpu.CompilerParams(needs_layout_passes=False),
    out_type=[
        jax.ShapeDtypeStruct((NW, L), jnp.float32),
        jax.ShapeDtypeStruct((NW, L), jnp.float32),
    ],
    scratch_types=[
        pltpu.VMEM((N_NODES,), jnp.int32),
        pltpu.VMEM((CHUNK_ROWS,), jnp.int32),
        pltpu.VMEM((CHUNK_EDGES,), jnp.int32),
        pltpu.VMEM((CHUNK_EDGES,), jnp.float32),
        pltpu.VMEM((L,), jnp.float32),
        pltpu.VMEM((L,), jnp.float32),
        pltpu.SemaphoreType.DMA,
    ],
)
def _smooth_loss_sc(src_hbm, tgt_hbm, ew_hbm, lab_hbm, out_s_hbm, out_r_hbm,
                    lab_v, src_v, tgt_v, ew_v, sacc_v, racc_v, sem):
    wid = lax.axis_index("s") * NC + lax.axis_index("c")
    pltpu.sync_copy(lab_hbm, lab_v)

    zeros = jnp.zeros((L,), jnp.float32)
    sacc_v[...] = zeros
    racc_v[...] = zeros
    lanes = lax.iota(jnp.int32, L)
    ninf = jnp.full((L,), -jnp.inf, jnp.float32)
    pinf = jnp.full((L,), jnp.inf, jnp.float32)

    def chunk_body(i, carry):
        c = wid + i * NW

        @pl.when(c < N_CHUNKS)
        def _():
            pltpu.sync_copy(src_hbm.at[pl.ds(c * CHUNK_ROWS, CHUNK_ROWS)],
                            src_v)
            pltpu.sync_copy(tgt_hbm.at[pl.ds(c * CHUNK_EDGES, CHUNK_EDGES)],
                            tgt_v)
            pltpu.sync_copy(ew_hbm.at[pl.ds(c * CHUNK_EDGES, CHUNK_EDGES)],
                            ew_v)

            def block_body(b, carry2):
                srow = src_v[pl.ds(b * ROWS_PER_BLOCK, L)]
                ls = plsc.load_gather(lab_v, [srow])
                cidx0 = b * EDGES_PER_BLOCK + lanes * K
                sp = zeros
                sn = zeros
                cp = zeros
                mp = ninf
                mn = pinf
                for j in range(K):
                    cidx = cidx0 + j
                    t = plsc.load_gather(tgt_v, [cidx])
                    lt = plsc.load_gather(lab_v, [t])
                    w = plsc.load_gather(ew_v, [cidx])
                    pos = ls == lt
                    sp = sp + jnp.where(pos, w, 0.0)
                    cp = cp + jnp.where(pos, 1.0, 0.0)
                    sn = sn + jnp.where(pos, 0.0, w)
                    mp = jnp.maximum(mp, jnp.where(pos, w, ninf))
                    mn = jnp.minimum(mn, jnp.where(pos, pinf, w))
                cn = jnp.float32(K) - cp
                term = jnp.exp(sp / cp) + jnp.exp(-(sn / cn))
                sacc_v[...] = sacc_v[...] + term
                racc_v[...] = racc_v[...] + jnp.maximum(0.0, mp - mn)
                return carry2

            lax.fori_loop(0, BLOCKS_PER_CHUNK, block_body, 0)
        return carry

    lax.fori_loop(0, MAX_CHUNKS_PER_W, chunk_body, 0)

    pltpu.sync_copy(sacc_v, out_s_hbm.at[wid])
    pltpu.sync_copy(racc_v, out_r_hbm.at[wid])


def kernel(edge_index, edge_weight, labels):
    src_rows = edge_index[0, ::K]
    tgt = edge_index[1]
    ew = edge_weight.reshape(-1)
    out_s, out_r = _smooth_loss_sc(src_rows, tgt, ew, labels)
    smoothness = out_s.sum()
    ranking = out_r.sum() / jnp.float32(N_ROWS)
    return jnp.log(smoothness + 2.0) + 2.0 * jnp.exp(ranking)


# speedup vs baseline: 481.9937x; 4.8681x over previous
import functools

import jax
import jax.numpy as jnp
from jax import lax
from jax.experimental import pallas as pl
from jax.experimental.pallas import tpu as pltpu
from jax.experimental.pallas import tpu_sc as plsc

K = 9
N_ROWS = 500000
N_EDGES = N_ROWS * K
N_NODES = 100000
L = 16
NC, NS = 2, 16
NW = NC * NS
ROWS_PER_BLOCK = L
EDGES_PER_BLOCK = ROWS_PER_BLOCK * K

CHUNK_EDGES = 4608
BLOCKS_PER_CHUNK = CHUNK_EDGES // EDGES_PER_BLOCK
N_FULL_CHUNKS = N_EDGES // CHUNK_EDGES
TAIL_EDGES = N_EDGES - N_FULL_CHUNKS * CHUNK_EDGES
TAIL_BLOCKS = TAIL_EDGES // EDGES_PER_BLOCK
N_CHUNKS = N_FULL_CHUNKS + 1
MAX_CHUNKS_PER_W = -(-N_CHUNKS // NW)

_mesh = plsc.VectorSubcoreMesh(core_axis_name="c", subcore_axis_name="s")


@functools.partial(
    pl.kernel,
    mesh=_mesh,
    compiler_params=pltpu.CompilerParams(needs_layout_passes=False),
    out_type=[
        jax.ShapeDtypeStruct((NW, L), jnp.float32),
        jax.ShapeDtypeStruct((NW, L), jnp.float32),
    ],
    scratch_types=[
        pltpu.VMEM((N_NODES,), jnp.int32),
        pltpu.VMEM((2, CHUNK_EDGES), jnp.int32),
        pltpu.VMEM((1, CHUNK_EDGES), jnp.float32),
        pltpu.VMEM((2, TAIL_EDGES), jnp.int32),
        pltpu.VMEM((1, TAIL_EDGES), jnp.float32),
        pltpu.VMEM((L,), jnp.float32),
        pltpu.VMEM((L,), jnp.float32),
        pltpu.SemaphoreType.DMA,
    ],
)
def _smooth_loss_sc(ei_hbm, ew_hbm, lab_hbm, tei_hbm, tew_hbm,
                    out_s_hbm, out_r_hbm,
                    lab_v, ei_v, ew_v, tei_v, tew_v, sacc_v, racc_v, sem):
    wid = lax.axis_index("s") * NC + lax.axis_index("c")
    pltpu.sync_copy(lab_hbm, lab_v)

    zeros = jnp.zeros((L,), jnp.float32)
    sacc_v[...] = zeros
    racc_v[...] = zeros
    lanes = lax.iota(jnp.int32, L)
    row0 = jnp.zeros((L,), jnp.int32)
    row1 = jnp.full((L,), 1, jnp.int32)
    ninf = jnp.full((L,), -jnp.inf, jnp.float32)
    pinf = jnp.full((L,), jnp.inf, jnp.float32)

    def process_blocks(nblocks, ei_v, ew_v):
        def block_body(b, carry2):
            cidx0 = b * EDGES_PER_BLOCK + lanes * K
            srow = plsc.load_gather(ei_v, [row0, cidx0])
            ls = plsc.load_gather(lab_v, [srow])
            sp = zeros
            sn = zeros
            cp = zeros
            mp = ninf
            mn = pinf
            for j in range(K):
                cidx = cidx0 + j
                t = plsc.load_gather(ei_v, [row1, cidx])
                lt = plsc.load_gather(lab_v, [t])
                w = plsc.load_gather(ew_v, [row0, cidx])
                pos = ls == lt
                sp = sp + jnp.where(pos, w, 0.0)
                cp = cp + jnp.where(pos, 1.0, 0.0)
                sn = sn + jnp.where(pos, 0.0, w)
                mp = jnp.maximum(mp, jnp.where(pos, w, ninf))
                mn = jnp.minimum(mn, jnp.where(pos, pinf, w))
            cn = jnp.float32(K) - cp
            term = jnp.exp(sp / cp) + jnp.exp(-(sn / cn))
            sacc_v[...] = sacc_v[...] + term
            racc_v[...] = racc_v[...] + jnp.maximum(0.0, mp - mn)
            return carry2

        lax.fori_loop(0, nblocks, block_body, 0)

    def chunk_body(i, carry):
        c = wid + i * NW

        @pl.when(c < N_FULL_CHUNKS)
        def _():
            base = c * CHUNK_EDGES
            pltpu.sync_copy(ei_hbm.at[:, pl.ds(base, CHUNK_EDGES)], ei_v)
            pltpu.sync_copy(ew_hbm.at[:, pl.ds(base, CHUNK_EDGES)], ew_v)
            process_blocks(BLOCKS_PER_CHUNK, ei_v, ew_v)

        @pl.when(c == N_FULL_CHUNKS)
        def _():
            pltpu.sync_copy(tei_hbm, tei_v)
            pltpu.sync_copy(tew_hbm, tew_v)
            process_blocks(TAIL_BLOCKS, tei_v, tew_v)

        return carry

    lax.fori_loop(0, MAX_CHUNKS_PER_W, chunk_body, 0)

    pltpu.sync_copy(sacc_v, out_s_hbm.at[wid])
    pltpu.sync_copy(racc_v, out_r_hbm.at[wid])


def kernel(edge_index, edge_weight, labels):
    tail_ei = lax.slice(edge_index, (0, N_FULL_CHUNKS * CHUNK_EDGES),
                        (2, N_EDGES))
    tail_ew = lax.slice(edge_weight, (0, N_FULL_CHUNKS * CHUNK_EDGES),
                        (1, N_EDGES))
    out_s, out_r = _smooth_loss_sc(edge_index, edge_weight, labels,
                                   tail_ei, tail_ew)
    smoothness = out_s.sum()
    ranking = out_r.sum() / jnp.float32(N_ROWS)
    return jnp.log(smoothness + 2.0) + 2.0 * jnp.exp(ranking)


# speedup vs baseline: 827.9764x; 1.7178x over previous
import functools

import jax
import jax.numpy as jnp
from jax import lax
from jax.experimental import pallas as pl
from jax.experimental.pallas import tpu as pltpu
from jax.experimental.pallas import tpu_sc as plsc

K = 9
N_ROWS = 500000
N_EDGES = N_ROWS * K
N_NODES = 100000
L = 16
NC, NS = 2, 16
NW = NC * NS
ROWS_PER_BLOCK = L
EDGES_PER_BLOCK = ROWS_PER_BLOCK * K

CHUNK_EDGES = 3456
BLOCKS_PER_CHUNK = CHUNK_EDGES // EDGES_PER_BLOCK
N_FULL_CHUNKS = N_EDGES // CHUNK_EDGES
TAIL_EDGES = N_EDGES - N_FULL_CHUNKS * CHUNK_EDGES
TAIL_BLOCKS = TAIL_EDGES // EDGES_PER_BLOCK
N_CHUNKS = N_FULL_CHUNKS + 1
MAX_CHUNKS_PER_W = -(-N_CHUNKS // NW)
MAX_I = -(-MAX_CHUNKS_PER_W // 2) * 2

_mesh = plsc.VectorSubcoreMesh(core_axis_name="c", subcore_axis_name="s")


@functools.partial(
    pl.kernel,
    mesh=_mesh,
    compiler_params=pltpu.CompilerParams(needs_layout_passes=False),
    out_type=[
        jax.ShapeDtypeStruct((NW, L), jnp.float32),
        jax.ShapeDtypeStruct((NW, L), jnp.float32),
    ],
    scratch_types=[
        pltpu.VMEM((N_NODES,), jnp.int32),
        pltpu.VMEM((2, CHUNK_EDGES), jnp.int32),
        pltpu.VMEM((1, CHUNK_EDGES), jnp.float32),
        pltpu.VMEM((2, CHUNK_EDGES), jnp.int32),
        pltpu.VMEM((1, CHUNK_EDGES), jnp.float32),
        pltpu.VMEM((2, TAIL_EDGES), jnp.int32),
        pltpu.VMEM((1, TAIL_EDGES), jnp.float32),
        pltpu.VMEM((L,), jnp.float32),
        pltpu.VMEM((L,), jnp.float32),
        pltpu.SemaphoreType.DMA,
        pltpu.SemaphoreType.DMA,
    ],
)
def _smooth_loss_sc(ei_hbm, ew_hbm, lab_hbm, tei_hbm, tew_hbm,
                    out_s_hbm, out_r_hbm,
                    lab_v, ei0_v, ew0_v, ei1_v, ew1_v, tei_v, tew_v,
                    sacc_v, racc_v, sem0, sem1):
    wid = lax.axis_index("s") * NC + lax.axis_index("c")
    pltpu.sync_copy(lab_hbm, lab_v)

    bufs = ((ei0_v, ew0_v, sem0), (ei1_v, ew1_v, sem1))
    zeros = jnp.zeros((L,), jnp.float32)
    sacc_v[...] = zeros
    racc_v[...] = zeros
    lanes9 = lax.iota(jnp.int32, L) * K
    row0 = jnp.zeros((L,), jnp.int32)
    row1 = jnp.full((L,), 1, jnp.int32)
    ninf = jnp.full((L,), -jnp.inf, jnp.float32)
    pinf = jnp.full((L,), jnp.inf, jnp.float32)

    def issue(c, p):
        ei_v, ew_v, sem = bufs[p]

        @pl.when(c < N_FULL_CHUNKS)
        def _():
            base = c * CHUNK_EDGES
            pltpu.async_copy(ei_hbm.at[:, pl.ds(base, CHUNK_EDGES)], ei_v,
                             sem)
            pltpu.async_copy(ew_hbm.at[:, pl.ds(base, CHUNK_EDGES)], ew_v,
                             sem)

    def wait(p):
        ei_v, ew_v, sem = bufs[p]
        pltpu.make_async_copy(ei_hbm.at[:, pl.ds(0, CHUNK_EDGES)], ei_v,
                              sem).wait()
        pltpu.make_async_copy(ew_hbm.at[:, pl.ds(0, CHUNK_EDGES)], ew_v,
                              sem).wait()

    def block_terms(b, ei_v, ew_v):
        cidx0 = b * EDGES_PER_BLOCK + lanes9
        srow = plsc.load_gather(ei_v, [row0, cidx0])
        ls = plsc.load_gather(lab_v, [srow])
        sp = zeros
        sn = zeros
        cp = zeros
        mp = ninf
        mn = pinf
        for j in range(K):
            cidx = cidx0 + j
            t = plsc.load_gather(ei_v, [row1, cidx])
            lt = plsc.load_gather(lab_v, [t])
            w = plsc.load_gather(ew_v, [row0, cidx])
            pos = ls == lt
            sp = sp + jnp.where(pos, w, 0.0)
            cp = cp + jnp.where(pos, 1.0, 0.0)
            sn = sn + jnp.where(pos, 0.0, w)
            mp = jnp.maximum(mp, jnp.where(pos, w, ninf))
            mn = jnp.minimum(mn, jnp.where(pos, pinf, w))
        cn = jnp.float32(K) - cp
        term = jnp.exp(sp / cp) + jnp.exp(-(sn / cn))
        rank = jnp.maximum(0.0, mp - mn)
        return term, rank

    def process_blocks(nblocks, ei_v, ew_v):
        def block_body(b2, carry):
            sacc, racc = carry
            for u in range(2):
                term, rank = block_terms(b2 * 2 + u, ei_v, ew_v)
                sacc = sacc + term
                racc = racc + rank
            return sacc, racc

        sacc, racc = lax.fori_loop(0, nblocks // 2, block_body,
                                   (zeros, zeros))
        sacc_v[...] = sacc_v[...] + sacc
        racc_v[...] = racc_v[...] + racc

    issue(wid, 0)

    def chunk_pair(i2, carry):
        for p in range(2):
            c = wid + (i2 * 2 + p) * NW
            cn = c + NW

            @pl.when(c < N_FULL_CHUNKS)
            def _():
                wait(p)
                issue(cn, 1 - p)
                ei_v, ew_v, _ = bufs[p]
                process_blocks(BLOCKS_PER_CHUNK, ei_v, ew_v)

            @pl.when(c == N_FULL_CHUNKS)
            def _():
                pltpu.sync_copy(tei_hbm, tei_v)
                pltpu.sync_copy(tew_hbm, tew_v)
                process_blocks(TAIL_BLOCKS, tei_v, tew_v)

        return carry

    lax.fori_loop(0, MAX_I // 2, chunk_pair, 0)

    pltpu.sync_copy(sacc_v, out_s_hbm.at[wid])
    pltpu.sync_copy(racc_v, out_r_hbm.at[wid])


def kernel(edge_index, edge_weight, labels):
    tail_ei = lax.slice(edge_index, (0, N_FULL_CHUNKS * CHUNK_EDGES),
                        (2, N_EDGES))
    tail_ew = lax.slice(edge_weight, (0, N_FULL_CHUNKS * CHUNK_EDGES),
                        (1, N_EDGES))
    out_s, out_r = _smooth_loss_sc(edge_index, edge_weight, labels,
                                   tail_ei, tail_ew)
    smoothness = out_s.sum()
    ranking = out_r.sum() / jnp.float32(N_ROWS)
    return jnp.log(smoothness + 2.0) + 2.0 * jnp.exp(ranking)
